# Initial kernel scaffold; baseline (speedup 1.0000x reference)
import jax
import jax.numpy as jnp
from jax.experimental import pallas as pl


def kernel(x, W_in, b_in, W_out, b_out):
    raise NotImplementedError("write your pallas kernel here")


# speedup vs baseline: 8.1909x; 8.1909x over previous
import functools

import jax
import jax.numpy as jnp
from jax.experimental import pallas as pl

_CODE_DIMS = 13
_LANE = 128
_ROW_BLOCK = 512


def _vq_kernel(x_ref, win_ref, bin_ref, wout_ref, bout_ref, out_ref, idx_ref):
    z = jnp.dot(x_ref[...], win_ref[...], preferred_element_type=jnp.float32)
    z = z + bin_ref[...]
    bits = (z > 0).astype(jnp.float32)
    q = bits * 2.0 - 1.0
    out_ref[...] = (
        jnp.dot(q, wout_ref[...], preferred_element_type=jnp.float32)
        + bout_ref[...]
    )
    col = jax.lax.broadcasted_iota(jnp.int32, z.shape, 1)
    w_idx = jnp.where(
        col < _CODE_DIMS,
        jnp.exp2((_CODE_DIMS - 1 - col).astype(jnp.float32)),
        0.0,
    )
    idx_ref[...] = jnp.sum(bits * w_idx, axis=1, keepdims=True).astype(jnp.int32)


@functools.partial(jax.jit, static_argnames=())
def kernel(x, W_in, b_in, W_out, b_out):
    B, T, DIM = x.shape
    n = B * T
    xf = x.reshape(n, DIM)

    win = jnp.zeros((DIM, _LANE), jnp.float32).at[:, :_CODE_DIMS].set(W_in)
    bin_ = jnp.zeros((1, _LANE), jnp.float32).at[0, :_CODE_DIMS].set(b_in)
    wout = jnp.zeros((_LANE, DIM), jnp.float32).at[:_CODE_DIMS, :].set(W_out)
    bout = b_out.reshape(1, DIM)

    grid = (n // _ROW_BLOCK,)
    out, idx = pl.pallas_call(
        _vq_kernel,
        grid=grid,
        in_specs=[
            pl.BlockSpec((_ROW_BLOCK, DIM), lambda i: (i, 0)),
            pl.BlockSpec((DIM, _LANE), lambda i: (0, 0)),
            pl.BlockSpec((1, _LANE), lambda i: (0, 0)),
            pl.BlockSpec((_LANE, DIM), lambda i: (0, 0)),
            pl.BlockSpec((1, DIM), lambda i: (0, 0)),
        ],
        out_specs=[
            pl.BlockSpec((_ROW_BLOCK, DIM), lambda i: (i, 0)),
            pl.BlockSpec((_ROW_BLOCK, 1), lambda i: (i, 0)),
        ],
        out_shape=[
            jax.ShapeDtypeStruct((n, DIM), jnp.float32),
            jax.ShapeDtypeStruct((n, 1), jnp.int32),
        ],
    )(xf, win, bin_, wout, bout)

    return out.reshape(B, T, DIM), idx.reshape(B, T)


# speedup vs baseline: 13.2244x; 1.6145x over previous
import jax
import jax.numpy as jnp
from jax.experimental import pallas as pl

_CODE_DIMS = 13


def _vq_kernel(x_ref, win_ref, bin_ref, wout_ref, bout_ref, out_ref, idx_ref):
    t, dim = x_ref.shape[1], x_ref.shape[2]
    x = x_ref[...].reshape(t, dim)
    z = jnp.dot(x, win_ref[...], preferred_element_type=jnp.float32)
    z = z + bin_ref[...]
    bits = (z > 0).astype(jnp.float32)
    q = bits * 2.0 - 1.0
    out = jnp.dot(q, wout_ref[...], preferred_element_type=jnp.float32)
    out_ref[...] = (out + bout_ref[...]).reshape(1, t, dim)
    col = jax.lax.broadcasted_iota(jnp.int32, (1, _CODE_DIMS), 1)
    w_idx = jnp.exp2((_CODE_DIMS - 1 - col).astype(jnp.float32))
    idx_f = jax.lax.dot_general(
        w_idx, bits, (((1,), (1,)), ((), ())),
        preferred_element_type=jnp.float32,
    )
    idx_ref[...] = idx_f.astype(jnp.int32).reshape(1, 1, t)


def kernel(x, W_in, b_in, W_out, b_out):
    B, T, DIM = x.shape
    out, idx = pl.pallas_call(
        _vq_kernel,
        grid=(B,),
        in_specs=[
            pl.BlockSpec((1, T, DIM), lambda i: (i, 0, 0)),
            pl.BlockSpec((DIM, _CODE_DIMS), lambda i: (0, 0)),
            pl.BlockSpec((1, _CODE_DIMS), lambda i: (0, 0)),
            pl.BlockSpec((_CODE_DIMS, DIM), lambda i: (0, 0)),
            pl.BlockSpec((1, DIM), lambda i: (0, 0)),
        ],
        out_specs=[
            pl.BlockSpec((1, T, DIM), lambda i: (i, 0, 0)),
            pl.BlockSpec((1, 1, T), lambda i: (i, 0, 0)),
        ],
        out_shape=[
            jax.ShapeDtypeStruct((B, T, DIM), jnp.float32),
            jax.ShapeDtypeStruct((B, 1, T), jnp.int32),
        ],
    )(x, W_in, b_in.reshape(1, _CODE_DIMS), W_out, b_out.reshape(1, DIM))

    return out, idx.reshape(B, T)


# speedup vs baseline: 14.9221x; 1.1284x over previous
import jax
import jax.numpy as jnp
from jax.experimental import pallas as pl

_CODE_DIMS = 13


def _vq_kernel(x_ref, win_ref, bin_ref, wout_ref, bout_ref, out_ref, idx_ref):
    t, dim = x_ref.shape[1], x_ref.shape[2]
    x = x_ref[...].reshape(t, dim)
    z = jnp.dot(x, win_ref[...], preferred_element_type=jnp.float32)
    z = z + bin_ref[...][None, :]
    bits = (z > 0).astype(jnp.float32)
    q = bits * 2.0 - 1.0
    out = jnp.dot(q, wout_ref[...], preferred_element_type=jnp.float32)
    out_ref[...] = (out + bout_ref[...][None, :]).reshape(1, t, dim)
    col = jax.lax.broadcasted_iota(jnp.int32, (1, _CODE_DIMS), 1)
    w_idx = jnp.exp2((_CODE_DIMS - 1 - col).astype(jnp.float32))
    idx_f = jax.lax.dot_general(
        w_idx, bits, (((1,), (1,)), ((), ())),
        preferred_element_type=jnp.float32,
    )
    i = pl.program_id(0)
    idx_ref[pl.ds(i, 1), :] = idx_f.astype(jnp.int32)


def kernel(x, W_in, b_in, W_out, b_out):
    B, T, DIM = x.shape
    out, idx = pl.pallas_call(
        _vq_kernel,
        grid=(B,),
        in_specs=[
            pl.BlockSpec((1, T, DIM), lambda i: (i, 0, 0)),
            pl.BlockSpec((DIM, _CODE_DIMS), lambda i: (0, 0)),
            pl.BlockSpec((_CODE_DIMS,), lambda i: (0,)),
            pl.BlockSpec((_CODE_DIMS, DIM), lambda i: (0, 0)),
            pl.BlockSpec((DIM,), lambda i: (0,)),
        ],
        out_specs=[
            pl.BlockSpec((1, T, DIM), lambda i: (i, 0, 0)),
            pl.BlockSpec((B, T), lambda i: (0, 0)),
        ],
        out_shape=[
            jax.ShapeDtypeStruct((B, T, DIM), jnp.float32),
            jax.ShapeDtypeStruct((B, T), jnp.int32),
        ],
    )(x, W_in, b_in, W_out, b_out)

    return out, idx


# speedup vs baseline: 17.5397x; 1.1754x over previous
import jax
import jax.numpy as jnp
from jax.experimental import pallas as pl

_CODE_DIMS = 13


def _vq_kernel(x_ref, win_ref, bin_ref, wout_ref, bout_ref, out_ref, idx_ref):
    t, dim = x_ref.shape[1], x_ref.shape[2]
    x = x_ref[...].reshape(t, dim)
    z = jax.lax.dot_general(
        x, win_ref[...], (((1,), (1,)), ((), ())),
        preferred_element_type=jnp.float32,
    )
    z = z + bin_ref[...][None, :]
    bits = (z > 0).astype(jnp.float32)
    q = bits * 2.0 - 1.0
    out = jnp.dot(q, wout_ref[...], preferred_element_type=jnp.float32)
    out_ref[...] = (out + bout_ref[...][None, :]).reshape(1, t, dim)
    col = jax.lax.broadcasted_iota(jnp.int32, (1, _CODE_DIMS), 1)
    w_idx = jnp.exp2((_CODE_DIMS - 1 - col).astype(jnp.float32))
    idx_f = jax.lax.dot_general(
        w_idx, bits, (((1,), (1,)), ((), ())),
        preferred_element_type=jnp.float32,
    )
    i = pl.program_id(0)
    idx_ref[pl.ds(i, 1), :] = idx_f.astype(jnp.int32)


def kernel(x, W_in, b_in, W_out, b_out):
    B, T, DIM = x.shape
    out, idx = pl.pallas_call(
        _vq_kernel,
        grid=(B,),
        in_specs=[
            pl.BlockSpec((1, T, DIM), lambda i: (i, 0, 0)),
            pl.BlockSpec((_CODE_DIMS, DIM), lambda i: (0, 0)),
            pl.BlockSpec((_CODE_DIMS,), lambda i: (0,)),
            pl.BlockSpec((_CODE_DIMS, DIM), lambda i: (0, 0)),
            pl.BlockSpec((DIM,), lambda i: (0,)),
        ],
        out_specs=[
            pl.BlockSpec((1, T, DIM), lambda i: (i, 0, 0)),
            pl.BlockSpec((B, T), lambda i: (0, 0)),
        ],
        out_shape=[
            jax.ShapeDtypeStruct((B, T, DIM), jnp.float32),
            jax.ShapeDtypeStruct((B, T), jnp.int32),
        ],
    )(x, W_in.T, b_in, W_out, b_out)

    return out, idx
